# baseline (speedup 1.0000x reference)
import functools

import jax
from jax import lax
import jax.numpy as jnp
from jax.experimental import pallas as pl
from jax.experimental.pallas import tpu as pltpu
from jax.experimental.pallas import tpu_sc as plsc

D_MODEL = 64
SCALE = 8.0
NRING = 8
GDEPTH = 4
LANES = 16
NW = 32


def kernel(x, table):
    x = x.astype(jnp.int32)
    b, s = x.shape
    rows_per_w = b // NW
    n = rows_per_w
    mesh = plsc.VectorSubcoreMesh(core_axis_name="c", subcore_axis_name="s")

    scratch = (
        [pltpu.VMEM((rows_per_w, s), jnp.int32)]
        + [pltpu.VMEM((s, D_MODEL), jnp.float32) for _ in range(NRING)]
        + [pltpu.SemaphoreType.DMA for _ in range(2 * NRING)]
    )

    @functools.partial(
        pl.kernel,
        out_type=jax.ShapeDtypeStruct((b * s, D_MODEL), table.dtype),
        mesh=mesh,
        scratch_types=scratch,
        compiler_params=pltpu.CompilerParams(use_tc_tiling_on_sc=False),
    )
    def gather_scale(table_hbm, idx_hbm, out_hbm, *bufs):
        idx_all = bufs[0]
        rows_v = bufs[1 : 1 + NRING]
        gsem = bufs[1 + NRING : 1 + 2 * NRING]
        osem = bufs[1 + 2 * NRING :]

        wid = lax.axis_index("c") * 16 + lax.axis_index("s")
        base = wid * rows_per_w

        def fire_gather(g, slot):
            pltpu.async_copy(table_hbm.at[idx_all.at[g]], rows_v[slot], gsem[slot])

        def wait_gather(g, slot):
            pltpu.make_async_copy(
                table_hbm.at[idx_all.at[g]], rows_v[slot], gsem[slot]
            ).wait()

        def scale_rows(slot):
            @pl.loop(0, s, step=4)
            def _row(t0):
                for dt in range(4):
                    for c in range(0, D_MODEL, LANES):
                        rows_v[slot][t0 + dt, pl.ds(c, LANES)] = (
                            rows_v[slot][t0 + dt, pl.ds(c, LANES)] * SCALE
                        )

        def fire_out(g, slot):
            pltpu.async_copy(
                rows_v[slot], out_hbm.at[pl.ds((base + g) * s, s)], osem[slot]
            )

        def wait_out(g, slot):
            pltpu.make_async_copy(
                rows_v[slot], out_hbm.at[pl.ds((base + g) * s, s)], osem[slot]
            ).wait()

        pltpu.sync_copy(idx_hbm.at[pl.ds(base, rows_per_w)], idx_all)

        for g in range(GDEPTH):
            fire_gather(g, g)

        for g in range(GDEPTH):
            wait_gather(g, g)
            scale_rows(g)
            fire_out(g, g)
            fire_gather(g + GDEPTH, g + GDEPTH)

        @pl.loop(GDEPTH, n - GDEPTH, step=NRING)
        def _main(g0):
            for k in range(NRING):
                g = g0 + k
                slot = (GDEPTH + k) % NRING
                wait_gather(g, slot)
                scale_rows(slot)
                fire_out(g, slot)
                wait_out(g - GDEPTH, k)
                fire_gather(g + GDEPTH, k)

        for k in range(GDEPTH):
            g = n - GDEPTH + k
            slot = (GDEPTH + k) % NRING
            wait_gather(g, slot)
            scale_rows(slot)
            fire_out(g, slot)
        for g in range(n - NRING, n):
            wait_out(g, g % NRING)

    return gather_scale(table, x).reshape(b, s, D_MODEL)
